# Initial kernel scaffold; baseline (speedup 1.0000x reference)
import jax
import jax.numpy as jnp
from jax.experimental import pallas as pl


def kernel(x, edge_index, x_prime, edge_index_prime, W1, b1, W2, b2, Wa1, ba1, Wa2, ba2, Wa3, ba3, Wa4, ba4, Wa5, ba5):
    raise NotImplementedError("write your pallas kernel here")


# speedup vs baseline: 29.6951x; 29.6951x over previous
import jax
import jax.numpy as jnp
from jax import lax
from jax.experimental import pallas as pl
from jax.experimental.pallas import tpu as pltpu
from jax.experimental.pallas import tpu_sc as plsc

N1 = 100000
N2 = 50000
NT = N1 + N2
NTP = 151552
E1 = 1600000
E2 = 800000
NS = 16
KB = 56
CPT = 1176
NBLK = CPT // KB
EPAD = NS * 128 * CPT
RZ = NTP // NS

BR = 1024
SEG_BD = 2000
SEG = N1 // SEG_BD
NBLKD = NT // SEG_BD


def _deg_body(dst2d, ones_hbm, zeros_hbm, degp, degacc, ones_v, idxbuf):
    s = lax.axis_index("s")
    r0 = s * RZ
    pltpu.sync_copy(zeros_hbm, degacc.at[pl.ds(r0, RZ), :])
    plsc.subcore_barrier()
    pltpu.sync_copy(ones_hbm, ones_v)
    base = s * CPT
    for t in range(NBLK):
        pltpu.sync_copy(dst2d.at[pl.ds(base + t * KB, KB), :], idxbuf)

        def body(j, carry):
            pltpu.sync_copy(ones_v, degacc.at[idxbuf.at[j]], add=True)
            return carry

        lax.fori_loop(0, KB, body, 0)
    plsc.subcore_barrier()
    pltpu.sync_copy(degacc.at[pl.ds(r0, RZ), :], degp.at[pl.ds(r0, RZ), :])


def _agg_body(src2d, dst2d, u_hbm, zeros_hbm, zp, zacc, idxs, idxd, gbuf, sem):
    s = lax.axis_index("s")
    r0 = s * RZ
    pltpu.sync_copy(zeros_hbm, zacc.at[pl.ds(r0, RZ), :])
    plsc.subcore_barrier()
    base = s * CPT
    for t in range(NBLK):
        pltpu.sync_copy(src2d.at[pl.ds(base + t * KB, KB), :], idxs)
        pltpu.sync_copy(dst2d.at[pl.ds(base + t * KB, KB), :], idxd)

        def body(j, carry):
            pltpu.async_copy(u_hbm.at[idxs.at[j]], gbuf, sem).wait()
            pltpu.sync_copy(gbuf, zacc.at[idxd.at[j]], add=True)
            return carry

        lax.fori_loop(0, KB, body, 0)
    plsc.subcore_barrier()
    pltpu.sync_copy(zacc.at[pl.ds(r0, RZ), :], zp.at[pl.ds(r0, RZ), :])


def _prep_body(degp_ref, x_ref, u_ref, dinv_ref):
    deg = degp_ref[:, 0:1] + 1.0
    dinv = lax.rsqrt(deg)
    dinv8 = jnp.broadcast_to(dinv, x_ref.shape)
    u_ref[...] = x_ref[...] * dinv8
    dinv_ref[...] = dinv8


def _head_body(z_ref, u_ref, dinv_ref, w1_ref, b1_ref, w2_ref, b2_ref,
               wa1_ref, ba1_ref, wa2_ref, ba2_ref, wa3_ref, ba3_ref,
               wa4_ref, ba4_ref, wa5_ref, ba5_ref, out_ref, acc1, acc2):
    i = pl.program_id(0)
    y = (z_ref[...] + u_ref[...]) * dinv_ref[...]

    @pl.when(i == 0)
    def _():
        acc1[...] = jnp.zeros_like(acc1)

    @pl.when(i == SEG)
    def _():
        acc2[...] = jnp.zeros_like(acc2)

    @pl.when(i < SEG)
    def _():
        h = jnp.dot(y, w1_ref[...], preferred_element_type=jnp.float32)
        h = jnp.maximum(h + b1_ref[...], 0.0)
        acc1[...] += jnp.sum(h, axis=0, keepdims=True)

    @pl.when(i >= SEG)
    def _():
        h = jnp.dot(y, w2_ref[...], preferred_element_type=jnp.float32)
        h = jnp.maximum(h + b2_ref[...], 0.0)
        acc2[...] += jnp.sum(h, axis=0, keepdims=True)

    @pl.when(i == NBLKD - 1)
    def _():
        g1 = acc1[...] * (1.0 / N1)
        g2 = acc2[...] * (1.0 / N2)
        a = (jnp.dot(g1, wa1_ref[0:128, :], preferred_element_type=jnp.float32)
             + jnp.dot(g2, wa1_ref[128:256, :], preferred_element_type=jnp.float32)
             + ba1_ref[...])
        a = jnp.maximum(a, 0.0)
        a = jnp.maximum(jnp.dot(a, wa2_ref[...], preferred_element_type=jnp.float32) + ba2_ref[...], 0.0)
        a = jnp.maximum(jnp.dot(a, wa3_ref[...], preferred_element_type=jnp.float32) + ba3_ref[...], 0.0)
        a = jnp.maximum(jnp.dot(a, wa4_ref[...], preferred_element_type=jnp.float32) + ba4_ref[...], 0.0)
        a = jnp.dot(a, wa5_ref[...], preferred_element_type=jnp.float32) + ba5_ref[...]
        m = jnp.max(a, axis=-1, keepdims=True)
        e = jnp.exp(a - m)
        out_ref[...] = e / jnp.sum(e, axis=-1, keepdims=True)


def kernel(x, edge_index, x_prime, edge_index_prime, W1, b1, W2, b2,
           Wa1, ba1, Wa2, ba2, Wa3, ba3, Wa4, ba4, Wa5, ba5):
    ei = edge_index.astype(jnp.int32)
    ep = edge_index_prime.astype(jnp.int32) + N1
    npad = EPAD - E1 - E2
    pad = jnp.full((npad,), NT, jnp.int32)
    src2d = jnp.concatenate([ei[0], ep[0], pad]).reshape(EPAD // 128, 128)
    dst2d = jnp.concatenate([ei[1], ep[1], pad]).reshape(EPAD // 128, 128)

    xc = jnp.concatenate([
        jnp.pad(x, ((0, 0), (0, 3))),
        x_prime,
        jnp.zeros((NTP - NT, 8), jnp.float32),
    ], axis=0)

    ones_hbm = jnp.ones((128, 8), jnp.float32)
    zeros_hbm = jnp.zeros((RZ, 8), jnp.float32)

    mesh = plsc.VectorSubcoreMesh(core_axis_name="c", subcore_axis_name="s",
                                  num_cores=1, num_subcores=NS)
    sc_params = pltpu.CompilerParams(use_tc_tiling_on_sc=False)

    degp = pl.kernel(
        _deg_body,
        out_type=jax.ShapeDtypeStruct((NTP, 8), jnp.float32),
        mesh=mesh,
        scratch_types=[
            pltpu.VMEM_SHARED((NTP, 8), jnp.float32),
            pltpu.VMEM((128, 8), jnp.float32),
            pltpu.VMEM((KB, 128), jnp.int32),
        ],
        compiler_params=sc_params,
    )(dst2d, ones_hbm, zeros_hbm)

    u, dinv8 = pl.pallas_call(
        _prep_body,
        grid=(NTP // BR,),
        in_specs=[
            pl.BlockSpec((BR, 8), lambda i: (i, 0)),
            pl.BlockSpec((BR, 8), lambda i: (i, 0)),
        ],
        out_specs=[
            pl.BlockSpec((BR, 8), lambda i: (i, 0)),
            pl.BlockSpec((BR, 8), lambda i: (i, 0)),
        ],
        out_shape=[
            jax.ShapeDtypeStruct((NTP, 8), jnp.float32),
            jax.ShapeDtypeStruct((NTP, 8), jnp.float32),
        ],
    )(degp, xc)

    zp = pl.kernel(
        _agg_body,
        out_type=jax.ShapeDtypeStruct((NTP, 8), jnp.float32),
        mesh=mesh,
        scratch_types=[
            pltpu.VMEM_SHARED((NTP, 8), jnp.float32),
            pltpu.VMEM((KB, 128), jnp.int32),
            pltpu.VMEM((KB, 128), jnp.int32),
            pltpu.VMEM((128, 8), jnp.float32),
            pltpu.SemaphoreType.DMA,
        ],
        compiler_params=sc_params,
    )(src2d, dst2d, u, zeros_hbm)

    w1p = jnp.pad(W1, ((0, 3), (0, 0)))

    full = lambda shape: pl.BlockSpec(shape, lambda i: tuple(0 for _ in shape))
    out = pl.pallas_call(
        _head_body,
        grid=(NBLKD,),
        in_specs=[
            pl.BlockSpec((SEG_BD, 8), lambda i: (i, 0)),
            pl.BlockSpec((SEG_BD, 8), lambda i: (i, 0)),
            pl.BlockSpec((SEG_BD, 8), lambda i: (i, 0)),
            full((8, 128)), full((1, 128)),
            full((8, 128)), full((1, 128)),
            full((256, 128)), full((1, 128)),
            full((128, 64)), full((1, 64)),
            full((64, 32)), full((1, 32)),
            full((32, 16)), full((1, 16)),
            full((16, 10)), full((1, 10)),
        ],
        out_specs=pl.BlockSpec((1, 10), lambda i: (0, 0)),
        out_shape=jax.ShapeDtypeStruct((1, 10), jnp.float32),
        scratch_shapes=[
            pltpu.VMEM((1, 128), jnp.float32),
            pltpu.VMEM((1, 128), jnp.float32),
        ],
    )(zp, u, dinv8, w1p, b1.reshape(1, 128), W2, b2.reshape(1, 128),
      Wa1, ba1.reshape(1, 128), Wa2, ba2.reshape(1, 64),
      Wa3, ba3.reshape(1, 32), Wa4, ba4.reshape(1, 16),
      Wa5, ba5.reshape(1, 10))
    return out


# speedup vs baseline: 31.7754x; 1.0701x over previous
import jax
import jax.numpy as jnp
from jax import lax
from jax.experimental import pallas as pl
from jax.experimental.pallas import tpu as pltpu
from jax.experimental.pallas import tpu_sc as plsc

N1 = 100000
N2 = 50000
NT = N1 + N2
NTP = 151552
E1 = 1600000
E2 = 800000
NS = 16
KB = 56
CPT = 1176
NBLK = CPT // KB
EPAD = NS * 128 * CPT
RZ = NTP // NS

BR = 1024
SEG_BD = 2000
SEG = N1 // SEG_BD
NBLKD = NT // SEG_BD


def _deg_body(dst2d, ones_hbm, zeros_hbm, degp, degacc, ones_v, idxbuf):
    s = lax.axis_index("s")
    r0 = s * RZ
    pltpu.sync_copy(zeros_hbm, degacc.at[pl.ds(r0, RZ), :])
    plsc.subcore_barrier()
    pltpu.sync_copy(ones_hbm, ones_v)
    base = s * CPT
    for t in range(NBLK):
        pltpu.sync_copy(dst2d.at[pl.ds(base + t * KB, KB), :], idxbuf)

        def body(j, carry):
            pltpu.sync_copy(ones_v, degacc.at[idxbuf.at[j]], add=True)
            return carry

        lax.fori_loop(0, KB, body, 0)
    plsc.subcore_barrier()
    pltpu.sync_copy(degacc.at[pl.ds(r0, RZ), :], degp.at[pl.ds(r0, RZ), :])


def _agg_body(src2d, dst2d, u_hbm, zeros_hbm, zp, zacc, idxs, idxd, gbuf, sem):
    s = lax.axis_index("s")
    r0 = s * RZ
    pltpu.sync_copy(zeros_hbm, zacc.at[pl.ds(r0, RZ), :])
    plsc.subcore_barrier()
    base = s * CPT
    for t in range(NBLK):
        pltpu.sync_copy(src2d.at[pl.ds(base + t * KB, KB), :], idxs)
        pltpu.sync_copy(dst2d.at[pl.ds(base + t * KB, KB), :], idxd)
        pltpu.async_copy(u_hbm.at[idxs.at[0]], gbuf.at[0], sem)

        def body(j, carry):
            b = lax.rem(j, 2)
            pltpu.make_async_copy(u_hbm.at[idxs.at[j]], gbuf.at[b], sem).wait()

            @pl.when(j + 1 < KB)
            def _():
                pltpu.async_copy(u_hbm.at[idxs.at[j + 1]], gbuf.at[1 - b], sem)

            pltpu.sync_copy(gbuf.at[b], zacc.at[idxd.at[j]], add=True)
            return carry

        lax.fori_loop(0, KB, body, 0)
    plsc.subcore_barrier()
    pltpu.sync_copy(zacc.at[pl.ds(r0, RZ), :], zp.at[pl.ds(r0, RZ), :])


def _prep_body(degp_ref, x_ref, u_ref, dinv_ref):
    deg = degp_ref[:, 0:1] + 1.0
    dinv = lax.rsqrt(deg)
    dinv8 = jnp.broadcast_to(dinv, x_ref.shape)
    u_ref[...] = x_ref[...] * dinv8
    dinv_ref[...] = dinv8


def _head_body(z_ref, u_ref, dinv_ref, w1_ref, b1_ref, w2_ref, b2_ref,
               wa1_ref, ba1_ref, wa2_ref, ba2_ref, wa3_ref, ba3_ref,
               wa4_ref, ba4_ref, wa5_ref, ba5_ref, out_ref, acc1, acc2):
    i = pl.program_id(0)
    y = (z_ref[...] + u_ref[...]) * dinv_ref[...]

    @pl.when(i == 0)
    def _():
        acc1[...] = jnp.zeros_like(acc1)

    @pl.when(i == SEG)
    def _():
        acc2[...] = jnp.zeros_like(acc2)

    @pl.when(i < SEG)
    def _():
        h = jnp.dot(y, w1_ref[...], preferred_element_type=jnp.float32)
        h = jnp.maximum(h + b1_ref[...], 0.0)
        acc1[...] += jnp.sum(h, axis=0, keepdims=True)

    @pl.when(i >= SEG)
    def _():
        h = jnp.dot(y, w2_ref[...], preferred_element_type=jnp.float32)
        h = jnp.maximum(h + b2_ref[...], 0.0)
        acc2[...] += jnp.sum(h, axis=0, keepdims=True)

    @pl.when(i == NBLKD - 1)
    def _():
        g1 = acc1[...] * (1.0 / N1)
        g2 = acc2[...] * (1.0 / N2)
        a = (jnp.dot(g1, wa1_ref[0:128, :], preferred_element_type=jnp.float32)
             + jnp.dot(g2, wa1_ref[128:256, :], preferred_element_type=jnp.float32)
             + ba1_ref[...])
        a = jnp.maximum(a, 0.0)
        a = jnp.maximum(jnp.dot(a, wa2_ref[...], preferred_element_type=jnp.float32) + ba2_ref[...], 0.0)
        a = jnp.maximum(jnp.dot(a, wa3_ref[...], preferred_element_type=jnp.float32) + ba3_ref[...], 0.0)
        a = jnp.maximum(jnp.dot(a, wa4_ref[...], preferred_element_type=jnp.float32) + ba4_ref[...], 0.0)
        a = jnp.dot(a, wa5_ref[...], preferred_element_type=jnp.float32) + ba5_ref[...]
        m = jnp.max(a, axis=-1, keepdims=True)
        e = jnp.exp(a - m)
        out_ref[...] = e / jnp.sum(e, axis=-1, keepdims=True)


def kernel(x, edge_index, x_prime, edge_index_prime, W1, b1, W2, b2,
           Wa1, ba1, Wa2, ba2, Wa3, ba3, Wa4, ba4, Wa5, ba5):
    ei = edge_index.astype(jnp.int32)
    ep = edge_index_prime.astype(jnp.int32) + N1
    npad = EPAD - E1 - E2
    pad = jnp.full((npad,), NT, jnp.int32)
    src2d = jnp.concatenate([ei[0], ep[0], pad]).reshape(EPAD // 128, 128)
    dst2d = jnp.concatenate([ei[1], ep[1], pad]).reshape(EPAD // 128, 128)

    xc = jnp.concatenate([
        jnp.pad(x, ((0, 0), (0, 3))),
        x_prime,
        jnp.zeros((NTP - NT, 8), jnp.float32),
    ], axis=0)

    ones_hbm = jnp.ones((128, 8), jnp.float32)
    zeros_hbm = jnp.zeros((RZ, 8), jnp.float32)

    mesh = plsc.VectorSubcoreMesh(core_axis_name="c", subcore_axis_name="s",
                                  num_cores=1, num_subcores=NS)
    sc_params = pltpu.CompilerParams(use_tc_tiling_on_sc=False)

    degp = pl.kernel(
        _deg_body,
        out_type=jax.ShapeDtypeStruct((NTP, 8), jnp.float32),
        mesh=mesh,
        scratch_types=[
            pltpu.VMEM_SHARED((NTP, 8), jnp.float32),
            pltpu.VMEM((128, 8), jnp.float32),
            pltpu.VMEM((KB, 128), jnp.int32),
        ],
        compiler_params=sc_params,
    )(dst2d, ones_hbm, zeros_hbm)

    u, dinv8 = pl.pallas_call(
        _prep_body,
        grid=(NTP // BR,),
        in_specs=[
            pl.BlockSpec((BR, 8), lambda i: (i, 0)),
            pl.BlockSpec((BR, 8), lambda i: (i, 0)),
        ],
        out_specs=[
            pl.BlockSpec((BR, 8), lambda i: (i, 0)),
            pl.BlockSpec((BR, 8), lambda i: (i, 0)),
        ],
        out_shape=[
            jax.ShapeDtypeStruct((NTP, 8), jnp.float32),
            jax.ShapeDtypeStruct((NTP, 8), jnp.float32),
        ],
    )(degp, xc)

    zp = pl.kernel(
        _agg_body,
        out_type=jax.ShapeDtypeStruct((NTP, 8), jnp.float32),
        mesh=mesh,
        scratch_types=[
            pltpu.VMEM_SHARED((NTP, 8), jnp.float32),
            pltpu.VMEM((KB, 128), jnp.int32),
            pltpu.VMEM((KB, 128), jnp.int32),
            pltpu.VMEM((2, 128, 8), jnp.float32),
            pltpu.SemaphoreType.DMA,
        ],
        compiler_params=sc_params,
    )(src2d, dst2d, u, zeros_hbm)

    w1p = jnp.pad(W1, ((0, 3), (0, 0)))

    full = lambda shape: pl.BlockSpec(shape, lambda i: tuple(0 for _ in shape))
    out = pl.pallas_call(
        _head_body,
        grid=(NBLKD,),
        in_specs=[
            pl.BlockSpec((SEG_BD, 8), lambda i: (i, 0)),
            pl.BlockSpec((SEG_BD, 8), lambda i: (i, 0)),
            pl.BlockSpec((SEG_BD, 8), lambda i: (i, 0)),
            full((8, 128)), full((1, 128)),
            full((8, 128)), full((1, 128)),
            full((256, 128)), full((1, 128)),
            full((128, 64)), full((1, 64)),
            full((64, 32)), full((1, 32)),
            full((32, 16)), full((1, 16)),
            full((16, 10)), full((1, 10)),
        ],
        out_specs=pl.BlockSpec((1, 10), lambda i: (0, 0)),
        out_shape=jax.ShapeDtypeStruct((1, 10), jnp.float32),
        scratch_shapes=[
            pltpu.VMEM((1, 128), jnp.float32),
            pltpu.VMEM((1, 128), jnp.float32),
        ],
    )(zp, u, dinv8, w1p, b1.reshape(1, 128), W2, b2.reshape(1, 128),
      Wa1, ba1.reshape(1, 128), Wa2, ba2.reshape(1, 64),
      Wa3, ba3.reshape(1, 32), Wa4, ba4.reshape(1, 16),
      Wa5, ba5.reshape(1, 10))
    return out


# speedup vs baseline: 48.4762x; 1.5256x over previous
import jax
import jax.numpy as jnp
from jax import lax
from jax.experimental import pallas as pl
from jax.experimental.pallas import tpu as pltpu
from jax.experimental.pallas import tpu_sc as plsc

N1 = 100000
N2 = 50000
NT = N1 + N2
NTP = 151552
E1 = 1600000
E2 = 800000
NS = 16
CHUNK = 512
KB = 40
CPT = 320
NBLK = CPT // KB
EPAD = NS * CHUNK * CPT
RZ = NTP // NS

BR = 1024
SEG_BD = 2000
SEG = N1 // SEG_BD
NBLKD = NT // SEG_BD


def _deg_body(dst2d, ones_hbm, zeros_hbm, degp, degacc, ones_v, idxbuf):
    s = lax.axis_index("s")
    r0 = s * RZ
    pltpu.sync_copy(zeros_hbm, degacc.at[pl.ds(r0, RZ), :])
    plsc.subcore_barrier()
    pltpu.sync_copy(ones_hbm, ones_v)
    base = s * CPT
    for t in range(NBLK):
        pltpu.sync_copy(dst2d.at[pl.ds(base + t * KB, KB), :], idxbuf)

        def body(j, carry):
            pltpu.sync_copy(ones_v, degacc.at[idxbuf.at[j]], add=True)
            return carry

        lax.fori_loop(0, KB, body, 0)
    plsc.subcore_barrier()
    pltpu.sync_copy(degacc.at[pl.ds(r0, RZ), :], degp.at[pl.ds(r0, RZ), :])


def _agg_body(src2d, dst2d, u_hbm, zeros_hbm, zp, zacc, idxs, idxd, gbuf, sem):
    s = lax.axis_index("s")
    r0 = s * RZ
    pltpu.sync_copy(zeros_hbm, zacc.at[pl.ds(r0, RZ), :])
    plsc.subcore_barrier()
    base = s * CPT
    for t in range(NBLK):
        pltpu.sync_copy(src2d.at[pl.ds(base + t * KB, KB), :], idxs)
        pltpu.sync_copy(dst2d.at[pl.ds(base + t * KB, KB), :], idxd)
        pltpu.async_copy(u_hbm.at[idxs.at[0]], gbuf.at[0], sem)

        def body(j, carry):
            b = lax.rem(j, 2)
            pltpu.make_async_copy(u_hbm.at[idxs.at[j]], gbuf.at[b], sem).wait()

            @pl.when(j + 1 < KB)
            def _():
                pltpu.async_copy(u_hbm.at[idxs.at[j + 1]], gbuf.at[1 - b], sem)

            pltpu.sync_copy(gbuf.at[b], zacc.at[idxd.at[j]], add=True)
            return carry

        lax.fori_loop(0, KB, body, 0)
    plsc.subcore_barrier()
    pltpu.sync_copy(zacc.at[pl.ds(r0, RZ), :], zp.at[pl.ds(r0, RZ), :])


def _prep_body(degp_ref, x_ref, u_ref, dinv_ref):
    deg = degp_ref[:, 0:1] + 1.0
    dinv = lax.rsqrt(deg)
    dinv8 = jnp.broadcast_to(dinv, x_ref.shape)
    u_ref[...] = x_ref[...] * dinv8
    dinv_ref[...] = dinv8


def _head_body(z_ref, u_ref, dinv_ref, w1_ref, b1_ref, w2_ref, b2_ref,
               wa1_ref, ba1_ref, wa2_ref, ba2_ref, wa3_ref, ba3_ref,
               wa4_ref, ba4_ref, wa5_ref, ba5_ref, out_ref, acc1, acc2):
    i = pl.program_id(0)
    y = (z_ref[...] + u_ref[...]) * dinv_ref[...]

    @pl.when(i == 0)
    def _():
        acc1[...] = jnp.zeros_like(acc1)

    @pl.when(i == SEG)
    def _():
        acc2[...] = jnp.zeros_like(acc2)

    @pl.when(i < SEG)
    def _():
        h = jnp.dot(y, w1_ref[...], preferred_element_type=jnp.float32)
        h = jnp.maximum(h + b1_ref[...], 0.0)
        acc1[...] += jnp.sum(h, axis=0, keepdims=True)

    @pl.when(i >= SEG)
    def _():
        h = jnp.dot(y, w2_ref[...], preferred_element_type=jnp.float32)
        h = jnp.maximum(h + b2_ref[...], 0.0)
        acc2[...] += jnp.sum(h, axis=0, keepdims=True)

    @pl.when(i == NBLKD - 1)
    def _():
        g1 = acc1[...] * (1.0 / N1)
        g2 = acc2[...] * (1.0 / N2)
        a = (jnp.dot(g1, wa1_ref[0:128, :], preferred_element_type=jnp.float32)
             + jnp.dot(g2, wa1_ref[128:256, :], preferred_element_type=jnp.float32)
             + ba1_ref[...])
        a = jnp.maximum(a, 0.0)
        a = jnp.maximum(jnp.dot(a, wa2_ref[...], preferred_element_type=jnp.float32) + ba2_ref[...], 0.0)
        a = jnp.maximum(jnp.dot(a, wa3_ref[...], preferred_element_type=jnp.float32) + ba3_ref[...], 0.0)
        a = jnp.maximum(jnp.dot(a, wa4_ref[...], preferred_element_type=jnp.float32) + ba4_ref[...], 0.0)
        a = jnp.dot(a, wa5_ref[...], preferred_element_type=jnp.float32) + ba5_ref[...]
        m = jnp.max(a, axis=-1, keepdims=True)
        e = jnp.exp(a - m)
        out_ref[...] = e / jnp.sum(e, axis=-1, keepdims=True)


def kernel(x, edge_index, x_prime, edge_index_prime, W1, b1, W2, b2,
           Wa1, ba1, Wa2, ba2, Wa3, ba3, Wa4, ba4, Wa5, ba5):
    ei = edge_index.astype(jnp.int32)
    ep = edge_index_prime.astype(jnp.int32) + N1
    npad = EPAD - E1 - E2
    pad = NT + jnp.arange(npad, dtype=jnp.int32) % 1024
    src2d = jnp.concatenate([ei[0], ep[0], pad]).reshape(EPAD // CHUNK, CHUNK)
    dst2d = jnp.concatenate([ei[1], ep[1], pad]).reshape(EPAD // CHUNK, CHUNK)

    xc = jnp.concatenate([
        jnp.pad(x, ((0, 0), (0, 3))),
        x_prime,
        jnp.zeros((NTP - NT, 8), jnp.float32),
    ], axis=0)

    ones_hbm = jnp.ones((CHUNK, 8), jnp.float32)
    zeros_hbm = jnp.zeros((RZ, 8), jnp.float32)

    mesh = plsc.VectorSubcoreMesh(core_axis_name="c", subcore_axis_name="s",
                                  num_cores=1, num_subcores=NS)
    sc_params = pltpu.CompilerParams(use_tc_tiling_on_sc=False)

    degp = pl.kernel(
        _deg_body,
        out_type=jax.ShapeDtypeStruct((NTP, 8), jnp.float32),
        mesh=mesh,
        scratch_types=[
            pltpu.VMEM_SHARED((NTP, 8), jnp.float32),
            pltpu.VMEM((CHUNK, 8), jnp.float32),
            pltpu.VMEM((KB, CHUNK), jnp.int32),
        ],
        compiler_params=sc_params,
    )(dst2d, ones_hbm, zeros_hbm)

    u, dinv8 = pl.pallas_call(
        _prep_body,
        grid=(NTP // BR,),
        in_specs=[
            pl.BlockSpec((BR, 8), lambda i: (i, 0)),
            pl.BlockSpec((BR, 8), lambda i: (i, 0)),
        ],
        out_specs=[
            pl.BlockSpec((BR, 8), lambda i: (i, 0)),
            pl.BlockSpec((BR, 8), lambda i: (i, 0)),
        ],
        out_shape=[
            jax.ShapeDtypeStruct((NTP, 8), jnp.float32),
            jax.ShapeDtypeStruct((NTP, 8), jnp.float32),
        ],
    )(degp, xc)

    zp = pl.kernel(
        _agg_body,
        out_type=jax.ShapeDtypeStruct((NTP, 8), jnp.float32),
        mesh=mesh,
        scratch_types=[
            pltpu.VMEM_SHARED((NTP, 8), jnp.float32),
            pltpu.VMEM((KB, CHUNK), jnp.int32),
            pltpu.VMEM((KB, CHUNK), jnp.int32),
            pltpu.VMEM((2, CHUNK, 8), jnp.float32),
            pltpu.SemaphoreType.DMA,
        ],
        compiler_params=sc_params,
    )(src2d, dst2d, u, zeros_hbm)

    w1p = jnp.pad(W1, ((0, 3), (0, 0)))

    full = lambda shape: pl.BlockSpec(shape, lambda i: tuple(0 for _ in shape))
    out = pl.pallas_call(
        _head_body,
        grid=(NBLKD,),
        in_specs=[
            pl.BlockSpec((SEG_BD, 8), lambda i: (i, 0)),
            pl.BlockSpec((SEG_BD, 8), lambda i: (i, 0)),
            pl.BlockSpec((SEG_BD, 8), lambda i: (i, 0)),
            full((8, 128)), full((1, 128)),
            full((8, 128)), full((1, 128)),
            full((256, 128)), full((1, 128)),
            full((128, 64)), full((1, 64)),
            full((64, 32)), full((1, 32)),
            full((32, 16)), full((1, 16)),
            full((16, 10)), full((1, 10)),
        ],
        out_specs=pl.BlockSpec((1, 10), lambda i: (0, 0)),
        out_shape=jax.ShapeDtypeStruct((1, 10), jnp.float32),
        scratch_shapes=[
            pltpu.VMEM((1, 128), jnp.float32),
            pltpu.VMEM((1, 128), jnp.float32),
        ],
    )(zp, u, dinv8, w1p, b1.reshape(1, 128), W2, b2.reshape(1, 128),
      Wa1, ba1.reshape(1, 128), Wa2, ba2.reshape(1, 64),
      Wa3, ba3.reshape(1, 32), Wa4, ba4.reshape(1, 16),
      Wa5, ba5.reshape(1, 10))
    return out


# speedup vs baseline: 63.0891x; 1.3014x over previous
import jax
import jax.numpy as jnp
from jax import lax
from jax.experimental import pallas as pl
from jax.experimental.pallas import tpu as pltpu
from jax.experimental.pallas import tpu_sc as plsc

N1 = 100000
N2 = 50000
NT = N1 + N2
NTP = 151552
E1 = 1600000
E2 = 800000
NS = 16
CHUNK = 512
KB = 40
CPT = 320
NBLK = CPT // KB
EPAD = NS * CHUNK * CPT
RZ = NTP // NS

BR = 1024
SEG_BD = 2000
SEG = N1 // SEG_BD
NBLKD = NT // SEG_BD


def _deg_body(dst2d, ones_hbm, zeros_hbm, degp, degacc, ones_v, idxbuf):
    s = lax.axis_index("s")
    r0 = s * RZ
    pltpu.sync_copy(zeros_hbm, degacc.at[pl.ds(r0, RZ), :])
    plsc.subcore_barrier()
    pltpu.sync_copy(ones_hbm, ones_v)
    base = s * CPT
    for t in range(NBLK):
        pltpu.sync_copy(dst2d.at[pl.ds(base + t * KB, KB), :], idxbuf)

        def body(j, carry):
            pltpu.sync_copy(ones_v, degacc.at[idxbuf.at[j]], add=True)
            return carry

        lax.fori_loop(0, KB, body, 0)
    plsc.subcore_barrier()
    pltpu.sync_copy(degacc.at[pl.ds(r0, RZ), :], degp.at[pl.ds(r0, RZ), :])


def _agg_body(src2d, dst2d, u_hbm, zeros_hbm, zp, zacc, u_sp, idxs, idxd, gbuf, sem):
    s = lax.axis_index("s")
    r0 = s * RZ
    pltpu.sync_copy(zeros_hbm, zacc.at[pl.ds(r0, RZ), :])
    pltpu.sync_copy(u_hbm.at[pl.ds(r0, RZ), :], u_sp.at[pl.ds(r0, RZ), :])
    plsc.subcore_barrier()
    base = s * CPT
    for t in range(NBLK):
        pltpu.sync_copy(src2d.at[pl.ds(base + t * KB, KB), :], idxs)
        pltpu.sync_copy(dst2d.at[pl.ds(base + t * KB, KB), :], idxd)
        pltpu.async_copy(u_sp.at[idxs.at[0]], gbuf.at[0], sem)

        def body(j, carry):
            b = lax.rem(j, 2)
            pltpu.make_async_copy(u_sp.at[idxs.at[j]], gbuf.at[b], sem).wait()

            @pl.when(j + 1 < KB)
            def _():
                pltpu.async_copy(u_sp.at[idxs.at[j + 1]], gbuf.at[1 - b], sem)

            pltpu.sync_copy(gbuf.at[b], zacc.at[idxd.at[j]], add=True)
            return carry

        lax.fori_loop(0, KB, body, 0)
    plsc.subcore_barrier()
    pltpu.sync_copy(zacc.at[pl.ds(r0, RZ), :], zp.at[pl.ds(r0, RZ), :])


def _prep_body(degp_ref, x_ref, u_ref, dinv_ref):
    deg = degp_ref[:, 0:1] + 1.0
    dinv = lax.rsqrt(deg)
    dinv8 = jnp.broadcast_to(dinv, x_ref.shape)
    u_ref[...] = (x_ref[...] * dinv8).astype(jnp.bfloat16)
    dinv_ref[...] = dinv8


def _head_body(z_ref, u_ref, dinv_ref, w1_ref, b1_ref, w2_ref, b2_ref,
               wa1_ref, ba1_ref, wa2_ref, ba2_ref, wa3_ref, ba3_ref,
               wa4_ref, ba4_ref, wa5_ref, ba5_ref, out_ref, acc1, acc2):
    i = pl.program_id(0)
    y = (z_ref[...].astype(jnp.float32)
         + u_ref[...].astype(jnp.float32)) * dinv_ref[...]

    @pl.when(i == 0)
    def _():
        acc1[...] = jnp.zeros_like(acc1)

    @pl.when(i == SEG)
    def _():
        acc2[...] = jnp.zeros_like(acc2)

    @pl.when(i < SEG)
    def _():
        h = jnp.dot(y, w1_ref[...], preferred_element_type=jnp.float32)
        h = jnp.maximum(h + b1_ref[...], 0.0)
        acc1[...] += jnp.sum(h, axis=0, keepdims=True)

    @pl.when(i >= SEG)
    def _():
        h = jnp.dot(y, w2_ref[...], preferred_element_type=jnp.float32)
        h = jnp.maximum(h + b2_ref[...], 0.0)
        acc2[...] += jnp.sum(h, axis=0, keepdims=True)

    @pl.when(i == NBLKD - 1)
    def _():
        g1 = acc1[...] * (1.0 / N1)
        g2 = acc2[...] * (1.0 / N2)
        a = (jnp.dot(g1, wa1_ref[0:128, :], preferred_element_type=jnp.float32)
             + jnp.dot(g2, wa1_ref[128:256, :], preferred_element_type=jnp.float32)
             + ba1_ref[...])
        a = jnp.maximum(a, 0.0)
        a = jnp.maximum(jnp.dot(a, wa2_ref[...], preferred_element_type=jnp.float32) + ba2_ref[...], 0.0)
        a = jnp.maximum(jnp.dot(a, wa3_ref[...], preferred_element_type=jnp.float32) + ba3_ref[...], 0.0)
        a = jnp.maximum(jnp.dot(a, wa4_ref[...], preferred_element_type=jnp.float32) + ba4_ref[...], 0.0)
        a = jnp.dot(a, wa5_ref[...], preferred_element_type=jnp.float32) + ba5_ref[...]
        m = jnp.max(a, axis=-1, keepdims=True)
        e = jnp.exp(a - m)
        out_ref[...] = e / jnp.sum(e, axis=-1, keepdims=True)


def kernel(x, edge_index, x_prime, edge_index_prime, W1, b1, W2, b2,
           Wa1, ba1, Wa2, ba2, Wa3, ba3, Wa4, ba4, Wa5, ba5):
    ei = edge_index.astype(jnp.int32)
    ep = edge_index_prime.astype(jnp.int32) + N1
    npad = EPAD - E1 - E2
    pad = NT + jnp.arange(npad, dtype=jnp.int32) % 1024
    src2d = jnp.concatenate([ei[0], ep[0], pad]).reshape(EPAD // CHUNK, CHUNK)
    dst2d = jnp.concatenate([ei[1], ep[1], pad]).reshape(EPAD // CHUNK, CHUNK)

    xc = jnp.concatenate([
        jnp.pad(x, ((0, 0), (0, 3))),
        x_prime,
        jnp.zeros((NTP - NT, 8), jnp.float32),
    ], axis=0)

    ones_hbm = jnp.ones((CHUNK, 8), jnp.float32)
    zeros_hbm = jnp.zeros((RZ, 8), jnp.float32)

    mesh = plsc.VectorSubcoreMesh(core_axis_name="c", subcore_axis_name="s",
                                  num_cores=1, num_subcores=NS)
    sc_params = pltpu.CompilerParams(use_tc_tiling_on_sc=False)

    degp = pl.kernel(
        _deg_body,
        out_type=jax.ShapeDtypeStruct((NTP, 8), jnp.float32),
        mesh=mesh,
        scratch_types=[
            pltpu.VMEM_SHARED((NTP, 8), jnp.float32),
            pltpu.VMEM((CHUNK, 8), jnp.float32),
            pltpu.VMEM((KB, CHUNK), jnp.int32),
        ],
        compiler_params=sc_params,
    )(dst2d, ones_hbm, zeros_hbm)

    u, dinv8 = pl.pallas_call(
        _prep_body,
        grid=(NTP // BR,),
        in_specs=[
            pl.BlockSpec((BR, 8), lambda i: (i, 0)),
            pl.BlockSpec((BR, 8), lambda i: (i, 0)),
        ],
        out_specs=[
            pl.BlockSpec((BR, 8), lambda i: (i, 0)),
            pl.BlockSpec((BR, 8), lambda i: (i, 0)),
        ],
        out_shape=[
            jax.ShapeDtypeStruct((NTP, 8), jnp.bfloat16),
            jax.ShapeDtypeStruct((NTP, 8), jnp.float32),
        ],
    )(degp, xc)

    zeros_bf = jnp.zeros((RZ, 8), jnp.bfloat16)
    zp = pl.kernel(
        _agg_body,
        out_type=jax.ShapeDtypeStruct((NTP, 8), jnp.bfloat16),
        mesh=mesh,
        scratch_types=[
            pltpu.VMEM_SHARED((NTP, 8), jnp.bfloat16),
            pltpu.VMEM_SHARED((NTP, 8), jnp.bfloat16),
            pltpu.VMEM((KB, CHUNK), jnp.int32),
            pltpu.VMEM((KB, CHUNK), jnp.int32),
            pltpu.VMEM((2, CHUNK, 8), jnp.bfloat16),
            pltpu.SemaphoreType.DMA,
        ],
        compiler_params=sc_params,
    )(src2d, dst2d, u, zeros_bf)

    w1p = jnp.pad(W1, ((0, 3), (0, 0)))

    full = lambda shape: pl.BlockSpec(shape, lambda i: tuple(0 for _ in shape))
    out = pl.pallas_call(
        _head_body,
        grid=(NBLKD,),
        in_specs=[
            pl.BlockSpec((SEG_BD, 8), lambda i: (i, 0)),
            pl.BlockSpec((SEG_BD, 8), lambda i: (i, 0)),
            pl.BlockSpec((SEG_BD, 8), lambda i: (i, 0)),
            full((8, 128)), full((1, 128)),
            full((8, 128)), full((1, 128)),
            full((256, 128)), full((1, 128)),
            full((128, 64)), full((1, 64)),
            full((64, 32)), full((1, 32)),
            full((32, 16)), full((1, 16)),
            full((16, 10)), full((1, 10)),
        ],
        out_specs=pl.BlockSpec((1, 10), lambda i: (0, 0)),
        out_shape=jax.ShapeDtypeStruct((1, 10), jnp.float32),
        scratch_shapes=[
            pltpu.VMEM((1, 128), jnp.float32),
            pltpu.VMEM((1, 128), jnp.float32),
        ],
    )(zp, u, dinv8, w1p, b1.reshape(1, 128), W2, b2.reshape(1, 128),
      Wa1, ba1.reshape(1, 128), Wa2, ba2.reshape(1, 64),
      Wa3, ba3.reshape(1, 32), Wa4, ba4.reshape(1, 16),
      Wa5, ba5.reshape(1, 10))
    return out


# speedup vs baseline: 63.0922x; 1.0000x over previous
import jax
import jax.numpy as jnp
from jax import lax
from jax.experimental import pallas as pl
from jax.experimental.pallas import tpu as pltpu
from jax.experimental.pallas import tpu_sc as plsc

N1 = 100000
N2 = 50000
NT = N1 + N2
NTP = 151552
E1 = 1600000
E2 = 800000
NS = 16
CHUNK = 512
KB = 40
CPT = 320
NBLK = CPT // KB
EPAD = NS * CHUNK * CPT
RZ = NTP // NS

BR = 1024
SEG_BD = 2000
SEG = N1 // SEG_BD
NBLKD = NT // SEG_BD


def _deg_body(dst2d, ones_hbm, zeros_hbm, degp, degacc, ones_v, idxbuf):
    s = lax.axis_index("s")
    r0 = s * RZ
    pltpu.sync_copy(zeros_hbm, degacc.at[pl.ds(r0, RZ), :])
    plsc.subcore_barrier()
    pltpu.sync_copy(ones_hbm, ones_v)
    base = s * CPT
    for t in range(NBLK):
        pltpu.sync_copy(dst2d.at[pl.ds(base + t * KB, KB), :], idxbuf)

        def body(j, carry):
            pltpu.sync_copy(ones_v, degacc.at[idxbuf.at[j]], add=True)
            return carry

        lax.fori_loop(0, KB, body, 0)
    plsc.subcore_barrier()
    pltpu.sync_copy(degacc.at[pl.ds(r0, RZ), :], degp.at[pl.ds(r0, RZ), :])


def _agg_body(src2d, dst2d, u_hbm, zeros_hbm, zp, zacc, u_sp, idxs, idxd, gbuf, sem):
    s = lax.axis_index("s")
    r0 = s * RZ
    pltpu.sync_copy(zeros_hbm, zacc.at[pl.ds(r0, RZ), :])
    pltpu.sync_copy(u_hbm.at[pl.ds(r0, RZ), :], u_sp.at[pl.ds(r0, RZ), :])
    plsc.subcore_barrier()
    base = s * CPT
    for t in range(NBLK):
        pltpu.sync_copy(src2d.at[pl.ds(base + t * KB, KB), :], idxs)
        pltpu.sync_copy(dst2d.at[pl.ds(base + t * KB, KB), :], idxd)
        pltpu.async_copy(u_sp.at[idxs.at[0]], gbuf.at[0], sem)

        def body(j, carry):
            b = lax.rem(j, 2)
            pltpu.make_async_copy(u_sp.at[idxs.at[j]], gbuf.at[b], sem).wait()

            @pl.when(j + 1 < KB)
            def _():
                pltpu.async_copy(u_sp.at[idxs.at[j + 1]], gbuf.at[1 - b], sem)

            pltpu.sync_copy(gbuf.at[b], zacc.at[idxd.at[j]], add=True)
            return carry

        lax.fori_loop(0, KB, body, 0)
    plsc.subcore_barrier()
    pltpu.sync_copy(zacc.at[pl.ds(r0, RZ), :], zp.at[pl.ds(r0, RZ), :])


def _prep_body(degp_ref, x_ref, u_ref, dinv_ref):
    deg = degp_ref[:, 0:1] + 1.0
    dinv = lax.rsqrt(deg)
    dinv8 = jnp.broadcast_to(dinv, x_ref.shape)
    u_ref[...] = (x_ref[...] * dinv8).astype(jnp.bfloat16)
    dinv_ref[...] = dinv8


def _head_body(z_ref, u_ref, dinv_ref, w1_ref, b1_ref, w2_ref, b2_ref,
               wa1_ref, ba1_ref, wa2_ref, ba2_ref, wa3_ref, ba3_ref,
               wa4_ref, ba4_ref, wa5_ref, ba5_ref, out_ref, acc1, acc2):
    i = pl.program_id(0)
    y = ((z_ref[...].astype(jnp.float32) + u_ref[...].astype(jnp.float32))
         * dinv_ref[...].astype(jnp.float32))

    @pl.when(i == 0)
    def _():
        acc1[...] = jnp.zeros_like(acc1)

    @pl.when(i == SEG)
    def _():
        acc2[...] = jnp.zeros_like(acc2)

    @pl.when(i < SEG)
    def _():
        h = jnp.dot(y, w1_ref[...], preferred_element_type=jnp.float32)
        h = jnp.maximum(h + b1_ref[...], 0.0)
        acc1[...] += jnp.sum(h, axis=0, keepdims=True)

    @pl.when(i >= SEG)
    def _():
        h = jnp.dot(y, w2_ref[...], preferred_element_type=jnp.float32)
        h = jnp.maximum(h + b2_ref[...], 0.0)
        acc2[...] += jnp.sum(h, axis=0, keepdims=True)

    @pl.when(i == NBLKD - 1)
    def _():
        g1 = acc1[...] * (1.0 / N1)
        g2 = acc2[...] * (1.0 / N2)
        a = (jnp.dot(g1, wa1_ref[0:128, :], preferred_element_type=jnp.float32)
             + jnp.dot(g2, wa1_ref[128:256, :], preferred_element_type=jnp.float32)
             + ba1_ref[...])
        a = jnp.maximum(a, 0.0)
        a = jnp.maximum(jnp.dot(a, wa2_ref[...], preferred_element_type=jnp.float32) + ba2_ref[...], 0.0)
        a = jnp.maximum(jnp.dot(a, wa3_ref[...], preferred_element_type=jnp.float32) + ba3_ref[...], 0.0)
        a = jnp.maximum(jnp.dot(a, wa4_ref[...], preferred_element_type=jnp.float32) + ba4_ref[...], 0.0)
        a = jnp.dot(a, wa5_ref[...], preferred_element_type=jnp.float32) + ba5_ref[...]
        m = jnp.max(a, axis=-1, keepdims=True)
        e = jnp.exp(a - m)
        out_ref[...] = e / jnp.sum(e, axis=-1, keepdims=True)


def kernel(x, edge_index, x_prime, edge_index_prime, W1, b1, W2, b2,
           Wa1, ba1, Wa2, ba2, Wa3, ba3, Wa4, ba4, Wa5, ba5):
    ei = edge_index.astype(jnp.int32)
    ep = edge_index_prime.astype(jnp.int32) + N1
    npad = EPAD - E1 - E2
    pad = NT + jnp.arange(npad, dtype=jnp.int32) % 1024
    src2d = jnp.concatenate([ei[0], ep[0], pad]).reshape(EPAD // CHUNK, CHUNK)
    dst2d = jnp.concatenate([ei[1], ep[1], pad]).reshape(EPAD // CHUNK, CHUNK)

    xc = jnp.concatenate([
        jnp.pad(x, ((0, 0), (0, 3))),
        x_prime,
        jnp.zeros((NTP - NT, 8), jnp.float32),
    ], axis=0)

    ones_hbm = jnp.ones((CHUNK, 8), jnp.float32)
    zeros_hbm = jnp.zeros((RZ, 8), jnp.float32)
    zeros_bf = jnp.zeros((RZ, 8), jnp.bfloat16)

    mesh = plsc.VectorSubcoreMesh(core_axis_name="c", subcore_axis_name="s",
                                  num_cores=1, num_subcores=NS)
    sc_params = pltpu.CompilerParams(use_tc_tiling_on_sc=False)

    degp = pl.kernel(
        _deg_body,
        out_type=jax.ShapeDtypeStruct((NTP, 8), jnp.float32),
        mesh=mesh,
        scratch_types=[
            pltpu.VMEM_SHARED((NTP, 8), jnp.float32),
            pltpu.VMEM((CHUNK, 8), jnp.float32),
            pltpu.VMEM((KB, CHUNK), jnp.int32),
        ],
        compiler_params=sc_params,
    )(dst2d, ones_hbm, zeros_hbm)

    u, dinv8 = pl.pallas_call(
        _prep_body,
        grid=(NTP // BR,),
        in_specs=[
            pl.BlockSpec((BR, 8), lambda i: (i, 0)),
            pl.BlockSpec((BR, 8), lambda i: (i, 0)),
        ],
        out_specs=[
            pl.BlockSpec((BR, 8), lambda i: (i, 0)),
            pl.BlockSpec((BR, 8), lambda i: (i, 0)),
        ],
        out_shape=[
            jax.ShapeDtypeStruct((NTP, 8), jnp.bfloat16),
            jax.ShapeDtypeStruct((NTP, 8), jnp.float32),
        ],
    )(degp, xc)

    zp = pl.kernel(
        _agg_body,
        out_type=jax.ShapeDtypeStruct((NTP, 8), jnp.bfloat16),
        mesh=mesh,
        scratch_types=[
            pltpu.VMEM_SHARED((NTP, 8), jnp.bfloat16),
            pltpu.VMEM_SHARED((NTP, 8), jnp.bfloat16),
            pltpu.VMEM((KB, CHUNK), jnp.int32),
            pltpu.VMEM((KB, CHUNK), jnp.int32),
            pltpu.VMEM((2, CHUNK, 8), jnp.bfloat16),
            pltpu.SemaphoreType.DMA,
        ],
        compiler_params=sc_params,
    )(src2d, dst2d, u, zeros_bf)

    w1p = jnp.pad(W1, ((0, 3), (0, 0)))

    full = lambda shape: pl.BlockSpec(shape, lambda i: tuple(0 for _ in shape))
    out = pl.pallas_call(
        _head_body,
        grid=(NBLKD,),
        in_specs=[
            pl.BlockSpec((SEG_BD, 8), lambda i: (i, 0)),
            pl.BlockSpec((SEG_BD, 8), lambda i: (i, 0)),
            pl.BlockSpec((SEG_BD, 8), lambda i: (i, 0)),
            full((8, 128)), full((1, 128)),
            full((8, 128)), full((1, 128)),
            full((256, 128)), full((1, 128)),
            full((128, 64)), full((1, 64)),
            full((64, 32)), full((1, 32)),
            full((32, 16)), full((1, 16)),
            full((16, 10)), full((1, 10)),
        ],
        out_specs=pl.BlockSpec((1, 10), lambda i: (0, 0)),
        out_shape=jax.ShapeDtypeStruct((1, 10), jnp.float32),
        scratch_shapes=[
            pltpu.VMEM((1, 128), jnp.float32),
            pltpu.VMEM((1, 128), jnp.float32),
        ],
    )(zp, u, dinv8, w1p, b1.reshape(1, 128), W2, b2.reshape(1, 128),
      Wa1, ba1.reshape(1, 128), Wa2, ba2.reshape(1, 64),
      Wa3, ba3.reshape(1, 32), Wa4, ba4.reshape(1, 16),
      Wa5, ba5.reshape(1, 10))
    return out
